# baseline (speedup 1.0000x reference)
import functools

import jax
import jax.numpy as jnp
from jax.experimental import pallas as pl
from jax.experimental.pallas import tpu as pltpu


def _st_kernel(x_ref, w0_ref, b0_ref, w1_ref, b1_ref, w2_ref, b2_ref, o_ref,
               wr_s, wm_s, w1_s, w2_s, b0_s, b1_s, b2_s, *, kernel_size):
    G, S, N = x_ref.shape
    GN = G * N
    pad = (kernel_size - 1) // 2

    @pl.when(pl.program_id(0) == 0)
    def _prep():
        i = jax.lax.broadcasted_iota(jnp.int32, (S, S), 0).astype(jnp.float32)
        j = jax.lax.broadcasted_iota(jnp.int32, (S, S), 1).astype(jnp.float32)
        band = (jnp.abs(i - j) <= pad).astype(jnp.float32)
        left = jnp.where(j == 0.0, jnp.maximum(pad - i, 0.0), 0.0)
        right = jnp.where(j == float(S - 1),
                          jnp.maximum(i - float(S - 1 - pad), 0.0), 0.0)
        A = (band + left + right) * (1.0 / kernel_size)

        w0 = w0_ref[...]
        wm = jnp.dot(w0, A, preferred_element_type=jnp.float32)
        wm_s[...] = (0.5 * wm).astype(jnp.bfloat16)
        wr_s[...] = (0.5 * (w0 - wm)).astype(jnp.bfloat16)
        b0_s[...] = 0.5 * b0_ref[...]
        w1 = w1_ref[...]
        w1_s[...] = (0.25 * w1).astype(jnp.bfloat16)
        b1_s[...] = 0.5 * b1_ref[...] + 0.25 * jnp.sum(w1, axis=1,
                                                       keepdims=True)
        w2 = w2_ref[...]
        w2_s[...] = (0.25 * w2).astype(jnp.bfloat16)
        b2_s[...] = 0.5 * b2_ref[...] + 0.25 * jnp.sum(w2, axis=1,
                                                       keepdims=True)

    xc = jnp.concatenate([x_ref[g] for g in range(G)],
                         axis=1).astype(jnp.bfloat16)

    b0 = b0_s[...]
    ut = jnp.tanh(jnp.dot(wr_s[...], xc,
                          preferred_element_type=jnp.float32) + b0)
    ur = jnp.tanh(jnp.dot(wm_s[...], xc,
                          preferred_element_type=jnp.float32) + b0)
    v = jnp.concatenate([ut, ur], axis=1).astype(jnp.bfloat16)
    u = jnp.tanh(jnp.dot(w1_s[...], v,
                         preferred_element_type=jnp.float32) + b1_s[...])
    u = u.astype(jnp.bfloat16)
    h = jnp.tanh(jnp.dot(w2_s[...], u,
                         preferred_element_type=jnp.float32) + b2_s[...])
    for g in range(G):
        o_ref[g] = (1.0 + 0.5 * (h[:, g * N:(g + 1) * N] +
                                 h[:, GN + g * N:GN + (g + 1) * N])
                    ).astype(o_ref.dtype)


def kernel(x, w0, b0, w1, b1, w2, b2):
    B, S, N = x.shape
    D = w0.shape[0]
    kernel_size = 25

    G = 8 if B % 8 == 0 else 1
    body = functools.partial(_st_kernel, kernel_size=kernel_size)
    out = pl.pallas_call(
        body,
        out_shape=jax.ShapeDtypeStruct((B, D, N), x.dtype),
        grid_spec=pltpu.PrefetchScalarGridSpec(
            num_scalar_prefetch=0,
            grid=(B // G,),
            in_specs=[
                pl.BlockSpec((G, S, N), lambda j: (j, 0, 0)),
                pl.BlockSpec((D, S), lambda j: (0, 0)),
                pl.BlockSpec((D, 1), lambda j: (0, 0)),
                pl.BlockSpec((D, D), lambda j: (0, 0)),
                pl.BlockSpec((D, 1), lambda j: (0, 0)),
                pl.BlockSpec((D, D), lambda j: (0, 0)),
                pl.BlockSpec((D, 1), lambda j: (0, 0)),
            ],
            out_specs=pl.BlockSpec((G, D, N), lambda j: (j, 0, 0)),
            scratch_shapes=[
                pltpu.VMEM((D, S), jnp.bfloat16),
                pltpu.VMEM((D, S), jnp.bfloat16),
                pltpu.VMEM((D, D), jnp.bfloat16),
                pltpu.VMEM((D, D), jnp.bfloat16),
                pltpu.VMEM((D, 1), jnp.float32),
                pltpu.VMEM((D, 1), jnp.float32),
                pltpu.VMEM((D, 1), jnp.float32),
            ],
        ),
        compiler_params=pltpu.CompilerParams(
            dimension_semantics=("arbitrary",)),
    )(x, w0, b0.reshape(D, 1), w1, b1.reshape(D, 1), w2, b2.reshape(D, 1))
    return out


# speedup vs baseline: 1.0125x; 1.0125x over previous
import functools

import jax
import jax.numpy as jnp
from jax.experimental import pallas as pl
from jax.experimental.pallas import tpu as pltpu


def _st_kernel(x_ref, w0_ref, b0_ref, w1_ref, b1_ref, w2_ref, b2_ref, o_ref,
               wr_s, wm_s, w1_s, w2_s, b0_s, b1_s, b2_s, *, kernel_size):
    G, S, N = x_ref.shape
    GN = G * N
    pad = (kernel_size - 1) // 2

    @pl.when(pl.program_id(0) == 0)
    def _prep():
        i = jax.lax.broadcasted_iota(jnp.int32, (S, S), 0).astype(jnp.float32)
        j = jax.lax.broadcasted_iota(jnp.int32, (S, S), 1).astype(jnp.float32)
        band = (jnp.abs(i - j) <= pad).astype(jnp.float32)
        left = jnp.where(j == 0.0, jnp.maximum(pad - i, 0.0), 0.0)
        right = jnp.where(j == float(S - 1),
                          jnp.maximum(i - float(S - 1 - pad), 0.0), 0.0)
        A = (band + left + right) * (1.0 / kernel_size)

        w0 = w0_ref[...]
        wm = jnp.dot(w0, A, preferred_element_type=jnp.float32)
        wm_s[...] = (0.5 * wm).astype(jnp.bfloat16)
        wr_s[...] = (0.5 * (w0 - wm)).astype(jnp.bfloat16)
        b0_s[...] = 0.5 * b0_ref[...]
        w1 = w1_ref[...]
        w1_s[...] = (0.25 * w1).astype(jnp.bfloat16)
        b1_s[...] = 0.5 * b1_ref[...] + 0.25 * jnp.sum(w1, axis=1,
                                                       keepdims=True)
        w2 = w2_ref[...]
        w2_s[...] = (0.25 * w2).astype(jnp.bfloat16)
        b2_s[...] = 0.5 * b2_ref[...] + 0.25 * jnp.sum(w2, axis=1,
                                                       keepdims=True)

    xc = jnp.concatenate([x_ref[g] for g in range(G)],
                         axis=1).astype(jnp.bfloat16)

    b0 = b0_s[...]
    ut = jnp.tanh(jnp.dot(wr_s[...], xc,
                          preferred_element_type=jnp.float32) + b0)
    ur = jnp.tanh(jnp.dot(wm_s[...], xc,
                          preferred_element_type=jnp.float32) + b0)
    v = jnp.concatenate([ut, ur], axis=1).astype(jnp.bfloat16)
    u = jnp.tanh(jnp.dot(w1_s[...], v,
                         preferred_element_type=jnp.float32) + b1_s[...])
    u = u.astype(jnp.bfloat16)
    h = jnp.tanh(jnp.dot(w2_s[...], u,
                         preferred_element_type=jnp.float32) + b2_s[...])
    for g in range(G):
        o_ref[g] = (1.0 + 0.5 * (h[:, g * N:(g + 1) * N] +
                                 h[:, GN + g * N:GN + (g + 1) * N])
                    ).astype(o_ref.dtype)


def kernel(x, w0, b0, w1, b1, w2, b2):
    B, S, N = x.shape
    D = w0.shape[0]
    kernel_size = 25

    G = 4 if B % 4 == 0 else 1
    body = functools.partial(_st_kernel, kernel_size=kernel_size)
    out = pl.pallas_call(
        body,
        out_shape=jax.ShapeDtypeStruct((B, D, N), x.dtype),
        grid_spec=pltpu.PrefetchScalarGridSpec(
            num_scalar_prefetch=0,
            grid=(B // G,),
            in_specs=[
                pl.BlockSpec((G, S, N), lambda j: (j, 0, 0)),
                pl.BlockSpec((D, S), lambda j: (0, 0)),
                pl.BlockSpec((D, 1), lambda j: (0, 0)),
                pl.BlockSpec((D, D), lambda j: (0, 0)),
                pl.BlockSpec((D, 1), lambda j: (0, 0)),
                pl.BlockSpec((D, D), lambda j: (0, 0)),
                pl.BlockSpec((D, 1), lambda j: (0, 0)),
            ],
            out_specs=pl.BlockSpec((G, D, N), lambda j: (j, 0, 0)),
            scratch_shapes=[
                pltpu.VMEM((D, S), jnp.bfloat16),
                pltpu.VMEM((D, S), jnp.bfloat16),
                pltpu.VMEM((D, D), jnp.bfloat16),
                pltpu.VMEM((D, D), jnp.bfloat16),
                pltpu.VMEM((D, 1), jnp.float32),
                pltpu.VMEM((D, 1), jnp.float32),
                pltpu.VMEM((D, 1), jnp.float32),
            ],
        ),
        compiler_params=pltpu.CompilerParams(
            dimension_semantics=("arbitrary",)),
    )(x, w0, b0.reshape(D, 1), w1, b1.reshape(D, 1), w2, b2.reshape(D, 1))
    return out


# speedup vs baseline: 1.1450x; 1.1308x over previous
import functools

import jax
import jax.numpy as jnp
from jax.experimental import pallas as pl
from jax.experimental.pallas import tpu as pltpu


def _st_kernel(x_ref, w0_ref, b0_ref, w1_ref, b1_ref, w2_ref, b2_ref, o_ref,
               wr_s, wm_s, w1_s, w2_s, b0_s, b1_s, b2_s, *, kernel_size):
    G, S, N = x_ref.shape
    GN = G * N
    pad = (kernel_size - 1) // 2

    @pl.when(pl.program_id(0) == 0)
    def _prep():
        i = jax.lax.broadcasted_iota(jnp.int32, (S, S), 0).astype(jnp.float32)
        j = jax.lax.broadcasted_iota(jnp.int32, (S, S), 1).astype(jnp.float32)
        band = (jnp.abs(i - j) <= pad).astype(jnp.float32)
        left = jnp.where(j == 0.0, jnp.maximum(pad - i, 0.0), 0.0)
        right = jnp.where(j == float(S - 1),
                          jnp.maximum(i - float(S - 1 - pad), 0.0), 0.0)
        A = (band + left + right) * (1.0 / kernel_size)

        w0 = w0_ref[...]
        wm = jnp.dot(w0, A, preferred_element_type=jnp.float32)
        wm_s[...] = (0.5 * wm).astype(jnp.bfloat16)
        wr_s[...] = (0.5 * (w0 - wm)).astype(jnp.bfloat16)
        b0_s[...] = 0.5 * jnp.reshape(b0_ref[...], (b0_ref.shape[0], 1))
        w1 = w1_ref[...]
        w1_s[...] = (0.25 * w1).astype(jnp.bfloat16)
        b1_s[...] = (0.5 * jnp.reshape(b1_ref[...], (b1_ref.shape[0], 1)) +
                     0.25 * jnp.sum(w1, axis=1, keepdims=True))
        w2 = w2_ref[...]
        w2_s[...] = (0.25 * w2).astype(jnp.bfloat16)
        b2_s[...] = (0.5 * jnp.reshape(b2_ref[...], (b2_ref.shape[0], 1)) +
                     0.25 * jnp.sum(w2, axis=1, keepdims=True))

    xc = jnp.concatenate([x_ref[g] for g in range(G)],
                         axis=1).astype(jnp.bfloat16)

    b0 = b0_s[...]
    ut = jnp.tanh(jnp.dot(wr_s[...], xc,
                          preferred_element_type=jnp.float32) + b0)
    ur = jnp.tanh(jnp.dot(wm_s[...], xc,
                          preferred_element_type=jnp.float32) + b0)
    v = jnp.concatenate([ut, ur], axis=1).astype(jnp.bfloat16)
    u = jnp.tanh(jnp.dot(w1_s[...], v,
                         preferred_element_type=jnp.float32) + b1_s[...])
    u = u.astype(jnp.bfloat16)
    h = jnp.tanh(jnp.dot(w2_s[...], u,
                         preferred_element_type=jnp.float32) + b2_s[...])
    for g in range(G):
        o_ref[g] = (1.0 + 0.5 * (h[:, g * N:(g + 1) * N] +
                                 h[:, GN + g * N:GN + (g + 1) * N])
                    ).astype(o_ref.dtype)


def kernel(x, w0, b0, w1, b1, w2, b2):
    B, S, N = x.shape
    D = w0.shape[0]
    kernel_size = 25

    G = 4 if B % 4 == 0 else 1
    body = functools.partial(_st_kernel, kernel_size=kernel_size)
    out = pl.pallas_call(
        body,
        out_shape=jax.ShapeDtypeStruct((B, D, N), x.dtype),
        grid_spec=pltpu.PrefetchScalarGridSpec(
            num_scalar_prefetch=0,
            grid=(B // G,),
            in_specs=[
                pl.BlockSpec((G, S, N), lambda j: (j, 0, 0)),
                pl.BlockSpec((D, S), lambda j: (0, 0)),
                pl.BlockSpec((D,), lambda j: (0,)),
                pl.BlockSpec((D, D), lambda j: (0, 0)),
                pl.BlockSpec((D,), lambda j: (0,)),
                pl.BlockSpec((D, D), lambda j: (0, 0)),
                pl.BlockSpec((D,), lambda j: (0,)),
            ],
            out_specs=pl.BlockSpec((G, D, N), lambda j: (j, 0, 0)),
            scratch_shapes=[
                pltpu.VMEM((D, S), jnp.bfloat16),
                pltpu.VMEM((D, S), jnp.bfloat16),
                pltpu.VMEM((D, D), jnp.bfloat16),
                pltpu.VMEM((D, D), jnp.bfloat16),
                pltpu.VMEM((D, 1), jnp.float32),
                pltpu.VMEM((D, 1), jnp.float32),
                pltpu.VMEM((D, 1), jnp.float32),
            ],
        ),
        compiler_params=pltpu.CompilerParams(
            dimension_semantics=("arbitrary",)),
    )(x, w0, b0, w1, b1, w2, b2)
    return out
